# Initial kernel scaffold; baseline (speedup 1.0000x reference)
import jax
import jax.numpy as jnp
from jax.experimental import pallas as pl


def kernel(x, A_pos, A_neg, edge_index, W_in, b_in, Wg0, bg0, Wg1, bg1, Wo0, bo0, Wo1, bo1, We1, be1, We2, be2):
    raise NotImplementedError("write your pallas kernel here")


# speedup vs baseline: 1.8241x; 1.8241x over previous
import functools

import jax
import jax.numpy as jnp
from jax import lax
from jax.experimental import pallas as pl
from jax.experimental.pallas import tpu as pltpu
from jax.experimental.pallas import tpu_sc as plsc

_N, _E, _D, _H = 10000, 320000, 128, 64

_RBLK = 200
_XBLK = 2000
_EBLK = 3200
_NW = 32
_GCH = 80


def _h0_body(x_ref, wt_ref, b_ref, o_ref):
    o_ref[...] = jnp.tanh(
        jnp.dot(x_ref[...], wt_ref[...], preferred_element_type=jnp.float32)
        + b_ref[...])


def _layer_body(ap_ref, an_ref, h_ref, wgp_ref, wgn_ref, bg_ref, wo_ref,
                bo_ref, o_ref):
    h = h_ref[...]
    hp = jnp.dot(ap_ref[...], h, preferred_element_type=jnp.float32)
    hn = jnp.dot(an_ref[...], h, preferred_element_type=jnp.float32)
    gate = jax.nn.sigmoid(
        jnp.dot(hp, wgp_ref[...], preferred_element_type=jnp.float32)
        + jnp.dot(hn, wgn_ref[...], preferred_element_type=jnp.float32)
        + bg_ref[...])
    hmix = gate * hp + (1.0 - gate) * hn
    o_ref[...] = jnp.tanh(
        jnp.dot(hmix, wo_ref[...], preferred_element_type=jnp.float32)
        + bo_ref[...])


def _edge_body(hu_ref, hv_ref, w1t_ref, b1_ref, w2_ref, b2_ref, o_ref):
    hu = hu_ref[0]
    hv = hv_ref[0]
    feat = jnp.concatenate([hu, hv, jnp.abs(hu - hv), hu * hv], axis=1)
    hid = jnp.maximum(
        jnp.dot(feat, w1t_ref[...], preferred_element_type=jnp.float32)
        + b1_ref[...], 0.0)
    logits = jnp.sum(hid * w2_ref[...], axis=1) + b2_ref[0]
    o_ref[...] = logits.reshape(1, 1, _EBLK)


def _gather_rows(h, idx_flat):
    nper = idx_flat.shape[0] // _NW
    nit = nper // _GCH
    mesh = plsc.VectorSubcoreMesh(core_axis_name="c", subcore_axis_name="s")

    @functools.partial(
        pl.kernel,
        mesh=mesh,
        out_type=jax.ShapeDtypeStruct((idx_flat.shape[0], _H), jnp.float32),
        scratch_types=[
            pltpu.VMEM((nper,), jnp.int32),
            pltpu.VMEM((_GCH, _H), jnp.float32),
            pltpu.SemaphoreType.DMA,
        ],
        compiler_params=pltpu.CompilerParams(use_tc_tiling_on_sc=False),
    )
    def _gather_kernel(h_hbm, idx_hbm, out_hbm, idx_v, rows_v, sem):
        wid = lax.axis_index("s") * 2 + lax.axis_index("c")
        base = pl.multiple_of(wid * nper, 8)
        pltpu.sync_copy(idx_hbm.at[pl.ds(base, nper)], idx_v)

        def body(j, carry):
            off = pl.multiple_of(j * _GCH, 8)
            pltpu.async_copy(
                h_hbm.at[idx_v.at[pl.ds(off, _GCH)]], rows_v, sem).wait()
            pltpu.sync_copy(rows_v, out_hbm.at[pl.ds(base + off, _GCH)])
            return carry

        lax.fori_loop(0, nit, body, 0)

    return _gather_kernel(h, idx_flat)


def kernel(x, A_pos, A_neg, edge_index, W_in, b_in, Wg0, bg0, Wg1, bg1,
           Wo0, bo0, Wo1, bo1, We1, be1, We2, be2):
    f32 = jnp.float32

    h = pl.pallas_call(
        _h0_body,
        grid=(_N // _XBLK,),
        in_specs=[
            pl.BlockSpec((_XBLK, _D), lambda i: (i, 0)),
            pl.BlockSpec((_D, _H), lambda i: (0, 0)),
            pl.BlockSpec((1, _H), lambda i: (0, 0)),
        ],
        out_specs=pl.BlockSpec((_XBLK, _H), lambda i: (i, 0)),
        out_shape=jax.ShapeDtypeStruct((_N, _H), f32),
    )(x, W_in.T, b_in.reshape(1, _H))

    layer = pl.pallas_call(
        _layer_body,
        grid=(_N // _RBLK,),
        in_specs=[
            pl.BlockSpec((_RBLK, _N), lambda r: (r, 0)),
            pl.BlockSpec((_RBLK, _N), lambda r: (r, 0)),
            pl.BlockSpec((_N, _H), lambda r: (0, 0)),
            pl.BlockSpec((_H, _H), lambda r: (0, 0)),
            pl.BlockSpec((_H, _H), lambda r: (0, 0)),
            pl.BlockSpec((1, _H), lambda r: (0, 0)),
            pl.BlockSpec((_H, _H), lambda r: (0, 0)),
            pl.BlockSpec((1, _H), lambda r: (0, 0)),
        ],
        out_specs=pl.BlockSpec((_RBLK, _H), lambda r: (r, 0)),
        out_shape=jax.ShapeDtypeStruct((_N, _H), f32),
        compiler_params=pltpu.CompilerParams(
            dimension_semantics=("arbitrary",)),
    )
    for (Wg, bg, Wo, bo) in ((Wg0, bg0, Wo0, bo0), (Wg1, bg1, Wo1, bo1)):
        h = layer(A_pos, A_neg, h, Wg[:, :_H].T, Wg[:, _H:].T,
                  bg.reshape(1, _H), Wo.T, bo.reshape(1, _H))

    g = _gather_rows(h, edge_index.reshape(-1))
    g3 = g.reshape(2, _E, _H)

    logits = pl.pallas_call(
        _edge_body,
        grid=(_E // _EBLK,),
        in_specs=[
            pl.BlockSpec((1, _EBLK, _H), lambda e: (0, e, 0)),
            pl.BlockSpec((1, _EBLK, _H), lambda e: (1, e, 0)),
            pl.BlockSpec((4 * _H, _H), lambda e: (0, 0)),
            pl.BlockSpec((1, _H), lambda e: (0, 0)),
            pl.BlockSpec((1, _H), lambda e: (0, 0)),
            pl.BlockSpec(memory_space=pltpu.SMEM),
        ],
        out_specs=pl.BlockSpec((1, 1, _EBLK), lambda e: (e, 0, 0)),
        out_shape=jax.ShapeDtypeStruct((_E // _EBLK, 1, _EBLK), f32),
    )(g3, g3, We1.T, be1.reshape(1, _H), We2, be2)
    return logits.reshape(_E)
